# Initial kernel scaffold; baseline (speedup 1.0000x reference)
import jax
import jax.numpy as jnp
from jax.experimental import pallas as pl


def kernel(edge_feats, segment_ids, num_segments, W, b):
    raise NotImplementedError("write your pallas kernel here")


# speedup vs baseline: 1.8248x; 1.8248x over previous
import functools

import jax
import jax.numpy as jnp
from jax import lax
from jax.experimental import pallas as pl
from jax.experimental.pallas import tpu as pltpu
from jax.experimental.pallas import tpu_sc as plsc


def _edge_weights(x, W, b2):
    E, D = x.shape
    BE = 4000

    def body(x_ref, w_ref, b_ref, o_ref):
        s = jnp.dot(x_ref[...], w_ref[...], preferred_element_type=jnp.float32)
        o_ref[...] = jnp.tanh(s + b_ref[0, 0])

    return pl.pallas_call(
        body,
        grid=(E // BE,),
        in_specs=[
            pl.BlockSpec((BE, D), lambda i: (i, 0)),
            pl.BlockSpec((D, 1), lambda i: (0, 0)),
            pl.BlockSpec((1, 1), lambda i: (0, 0)),
        ],
        out_specs=pl.BlockSpec((BE, 1), lambda i: (i, 0)),
        out_shape=jax.ShapeDtypeStruct((E, 1), jnp.float32),
    )(x, W, b2)


@functools.cache
def _sc_segsum(E, D, G):
    NC, NS = 2, 16
    NW = NC * NS
    EPW = E // NW
    C = 400
    NCHUNK = EPW // C
    GD = G * D
    KD = D // 16

    mesh = plsc.VectorSubcoreMesh(core_axis_name="c", subcore_axis_name="s")

    @functools.partial(
        pl.kernel,
        mesh=mesh,
        out_type=jax.ShapeDtypeStruct((NW, GD), jnp.float32),
        scratch_types=[
            pltpu.VMEM((C * D,), jnp.float32),
            pltpu.VMEM((C,), jnp.int32),
            pltpu.VMEM((C,), jnp.float32),
            pltpu.VMEM((GD,), jnp.float32),
        ],
    )
    def body(x_hbm, seg_hbm, w_hbm, out_hbm, xbuf, segbuf, wbuf, acc):
        wid = lax.axis_index("c") * NS + lax.axis_index("s")
        base = wid * EPW
        z16 = jnp.zeros((16,), jnp.float32)

        def zero_body(i, c):
            acc[pl.ds(i * 16, 16)] = z16
            return c

        lax.fori_loop(0, GD // 16, zero_body, 0)

        def chunk_body(ci, c):
            cb = base + ci * C
            pltpu.sync_copy(x_hbm.at[pl.ds(cb * D, C * D)], xbuf)
            pltpu.sync_copy(seg_hbm.at[pl.ds(cb, C)], segbuf)
            pltpu.sync_copy(w_hbm.at[pl.ds(cb, C)], wbuf)

            def blk_body(eb, c2):
                e0 = eb * 16
                segv = jnp.minimum(segbuf[pl.ds(e0, 16)], G - 1)
                wv = wbuf[pl.ds(e0, 16)]
                for j in range(16):
                    off = segv[j] * D
                    eoff = (e0 + j) * D
                    wj = wv[j]
                    for k in range(KD):
                        xk = xbuf[pl.ds(eoff + k * 16, 16)]
                        plsc.addupdate(acc.at[pl.ds(off + k * 16, 16)], xk * wj)
                return c2

            lax.fori_loop(0, C // 16, blk_body, 0)
            return c

        lax.fori_loop(0, NCHUNK, chunk_body, 0)
        pltpu.sync_copy(acc, out_hbm.at[wid])

    return body


def _reduce_partials(p, G, D):
    NW = p.shape[0]

    def body(p_ref, o_ref):
        o_ref[...] = jnp.sum(p_ref[...], axis=0, keepdims=True)

    return pl.pallas_call(
        body,
        out_shape=jax.ShapeDtypeStruct((1, G * D), jnp.float32),
    )(p)


def kernel(edge_feats, segment_ids, num_segments, W, b):
    E, D = edge_feats.shape
    G = 256
    w = _edge_weights(edge_feats, W, b.reshape(1, 1))
    partials = _sc_segsum(E, D, G)(
        edge_feats.reshape(-1), segment_ids, w.reshape(-1)
    )
    h_g_sum = _reduce_partials(partials, G, D).reshape(G, D)
    return (h_g_sum, w)


# speedup vs baseline: 2.0943x; 1.1477x over previous
import functools

import jax
import jax.numpy as jnp
from jax import lax
from jax.experimental import pallas as pl
from jax.experimental.pallas import tpu as pltpu
from jax.experimental.pallas import tpu_sc as plsc


def _edge_weights(x, W, b2):
    E, D = x.shape
    BE = 4000

    def body(x_ref, w_ref, b_ref, o_ref):
        s = jnp.dot(x_ref[...], w_ref[...], preferred_element_type=jnp.float32)
        o_ref[...] = jnp.tanh(s + b_ref[0, 0])

    return pl.pallas_call(
        body,
        grid=(E // BE,),
        in_specs=[
            pl.BlockSpec((BE, D), lambda i: (i, 0)),
            pl.BlockSpec((D, 1), lambda i: (0, 0)),
            pl.BlockSpec((1, 1), lambda i: (0, 0)),
        ],
        out_specs=pl.BlockSpec((BE, 1), lambda i: (i, 0)),
        out_shape=jax.ShapeDtypeStruct((E, 1), jnp.float32),
    )(x, W, b2)


@functools.cache
def _sc_segsum(E, D, G):
    NC, NS = 2, 16
    NW = NC * NS
    EPW = E // NW
    SUB = 80
    NBUF = 5
    NSUB = EPW // SUB
    NOUT = NSUB // NBUF
    GD = G * D
    KD = D // 16

    mesh = plsc.VectorSubcoreMesh(core_axis_name="c", subcore_axis_name="s")

    @functools.partial(
        pl.kernel,
        mesh=mesh,
        out_type=jax.ShapeDtypeStruct((NW, GD), jnp.float32),
        scratch_types=[
            pltpu.VMEM((NBUF * SUB * D,), jnp.float32),
            pltpu.VMEM((NBUF * SUB,), jnp.int32),
            pltpu.VMEM((NBUF * SUB,), jnp.float32),
            pltpu.VMEM((GD,), jnp.float32),
            pltpu.SemaphoreType.DMA,
            pltpu.SemaphoreType.DMA,
            pltpu.SemaphoreType.DMA,
            pltpu.SemaphoreType.DMA,
            pltpu.SemaphoreType.DMA,
        ],
    )
    def body(x_hbm, seg_hbm, w_hbm, out_hbm, xbuf, segb, wbb, acc,
             s0, s1, s2, s3, s4):
        sems = (s0, s1, s2, s3, s4)
        wid = lax.axis_index("c") * NS + lax.axis_index("s")
        base = wid * EPW
        z16 = jnp.zeros((16,), jnp.float32)

        def zero_body(i, c):
            acc[pl.ds(i * 16, 16)] = z16
            return c

        lax.fori_loop(0, GD // 16, zero_body, 0)

        def issue(ci, b):
            cb = base + ci * SUB
            pltpu.async_copy(x_hbm.at[pl.ds(cb * D, SUB * D)],
                             xbuf.at[pl.ds(b * SUB * D, SUB * D)], sems[b])
            pltpu.async_copy(seg_hbm.at[pl.ds(cb, SUB)],
                             segb.at[pl.ds(b * SUB, SUB)], sems[b])
            pltpu.async_copy(w_hbm.at[pl.ds(cb, SUB)],
                             wbb.at[pl.ds(b * SUB, SUB)], sems[b])

        def drain(b):
            pltpu.make_async_copy(x_hbm.at[pl.ds(0, SUB * D)],
                                  xbuf.at[pl.ds(b * SUB * D, SUB * D)],
                                  sems[b]).wait()
            pltpu.make_async_copy(seg_hbm.at[pl.ds(0, SUB)],
                                  segb.at[pl.ds(b * SUB, SUB)], sems[b]).wait()
            pltpu.make_async_copy(w_hbm.at[pl.ds(0, SUB)],
                                  wbb.at[pl.ds(b * SUB, SUB)], sems[b]).wait()

        for b in range(NBUF):
            issue(b, b)

        def outer(cg, c):
            for b in range(NBUF):
                ci = cg * NBUF + b
                drain(b)

                def blk(bi, c2, _b=b):
                    e0 = _b * SUB + bi * 16
                    segv = jnp.minimum(segb[pl.ds(e0, 16)], G - 1)
                    wv = wbb[pl.ds(e0, 16)]
                    offv = segv * D
                    for j in range(16):
                        off = offv[j]
                        wj = wv[j]
                        eoff = (e0 + j) * D
                        for k in range(KD):
                            xk = xbuf[pl.ds(eoff + k * 16, 16)]
                            plsc.addupdate(acc.at[pl.ds(off + k * 16, 16)],
                                           xk * wj)
                    return c2

                lax.fori_loop(0, SUB // 16, blk, 0)

                nci = ci + NBUF

                @pl.when(nci < NSUB)
                def _(b=b, nci=nci):
                    issue(nci, b)

            return c

        lax.fori_loop(0, NOUT, outer, 0)
        pltpu.sync_copy(acc, out_hbm.at[wid])

    return body


def _reduce_partials(p, G, D):
    NW = p.shape[0]

    def body(p_ref, o_ref):
        o_ref[...] = jnp.sum(p_ref[...], axis=0, keepdims=True)

    return pl.pallas_call(
        body,
        out_shape=jax.ShapeDtypeStruct((1, G * D), jnp.float32),
    )(p)


def kernel(edge_feats, segment_ids, num_segments, W, b):
    E, D = edge_feats.shape
    G = 256
    w = _edge_weights(edge_feats, W, b.reshape(1, 1))
    partials = _sc_segsum(E, D, G)(
        edge_feats.reshape(-1), segment_ids, w.reshape(-1)
    )
    h_g_sum = _reduce_partials(partials, G, D).reshape(G, D)
    return (h_g_sum, w)


# speedup vs baseline: 3.9808x; 1.9007x over previous
import functools

import jax
import jax.numpy as jnp
from jax import lax
from jax.experimental import pallas as pl
from jax.experimental.pallas import tpu as pltpu
from jax.experimental.pallas import tpu_sc as plsc


def _edge_weights(x, W, b2):
    E, D = x.shape
    BE = 4000

    def body(x_ref, w_ref, b_ref, o_ref):
        s = jnp.dot(x_ref[...], w_ref[...], preferred_element_type=jnp.float32)
        o_ref[...] = jnp.tanh(s + b_ref[0, 0])

    return pl.pallas_call(
        body,
        grid=(E // BE,),
        in_specs=[
            pl.BlockSpec((BE, D), lambda i: (i, 0)),
            pl.BlockSpec((D, 1), lambda i: (0, 0)),
            pl.BlockSpec((1, 1), lambda i: (0, 0)),
        ],
        out_specs=pl.BlockSpec((BE, 1), lambda i: (i, 0)),
        out_shape=jax.ShapeDtypeStruct((E, 1), jnp.float32),
    )(x, W, b2)


@functools.cache
def _sc_segsum(E, D, G):
    NC, NS = 2, 16
    NW = NC * NS
    EPW = E // NW
    SUB = 80
    NBUF = 5
    NSUB = EPW // SUB
    NOUT = NSUB // NBUF
    GD = G * D
    KD = D // 16

    mesh = plsc.VectorSubcoreMesh(core_axis_name="c", subcore_axis_name="s")

    @functools.partial(
        pl.kernel,
        mesh=mesh,
        out_type=jax.ShapeDtypeStruct((NW, GD), jnp.float32),
        scratch_types=[
            pltpu.VMEM((NBUF * SUB * D,), jnp.float32),
            pltpu.VMEM((NBUF * SUB,), jnp.int32),
            pltpu.VMEM((NBUF * SUB,), jnp.float32),
            pltpu.VMEM((GD,), jnp.float32),
            pltpu.SemaphoreType.DMA,
            pltpu.SemaphoreType.DMA,
            pltpu.SemaphoreType.DMA,
            pltpu.SemaphoreType.DMA,
            pltpu.SemaphoreType.DMA,
        ],
    )
    def body(x_hbm, seg_hbm, w_hbm, out_hbm, xbuf, segb, wbb, acc,
             s0, s1, s2, s3, s4):
        sems = (s0, s1, s2, s3, s4)
        wid = lax.axis_index("c") * NS + lax.axis_index("s")
        base = wid * EPW
        z16 = jnp.zeros((16,), jnp.float32)
        lane = [jnp.full((16,), j, jnp.int32) for j in range(16)]

        def zero_body(i, c):
            acc[pl.ds(i * 16, 16)] = z16
            return c

        lax.fori_loop(0, GD // 16, zero_body, 0)

        def issue(ci, b):
            cb = base + ci * SUB
            pltpu.async_copy(x_hbm.at[pl.ds(cb * D, SUB * D)],
                             xbuf.at[pl.ds(b * SUB * D, SUB * D)], sems[b])
            pltpu.async_copy(seg_hbm.at[pl.ds(cb, SUB)],
                             segb.at[pl.ds(b * SUB, SUB)], sems[b])
            pltpu.async_copy(w_hbm.at[pl.ds(cb, SUB)],
                             wbb.at[pl.ds(b * SUB, SUB)], sems[b])

        def drain(b):
            pltpu.make_async_copy(x_hbm.at[pl.ds(0, SUB * D)],
                                  xbuf.at[pl.ds(b * SUB * D, SUB * D)],
                                  sems[b]).wait()
            pltpu.make_async_copy(seg_hbm.at[pl.ds(0, SUB)],
                                  segb.at[pl.ds(b * SUB, SUB)], sems[b]).wait()
            pltpu.make_async_copy(w_hbm.at[pl.ds(0, SUB)],
                                  wbb.at[pl.ds(b * SUB, SUB)], sems[b]).wait()

        for b in range(NBUF):
            issue(b, b)

        def outer(cg, c):
            for b in range(NBUF):
                ci = cg * NBUF + b
                drain(b)

                def blk(bi, c2, _b=b):
                    e0 = _b * SUB + bi * 16
                    segv = jnp.minimum(segb[pl.ds(e0, 16)], G - 1)
                    wv = wbb[pl.ds(e0, 16)]
                    s_first = segv[0]
                    s_last = segv[15]

                    @pl.when(s_first == s_last)
                    def _():
                        accs = [jnp.zeros((16,), jnp.float32)
                                for _ in range(KD)]
                        for j in range(16):
                            wjv = jnp.take_along_axis(wv, lane[j], axis=0)
                            eoff = (e0 + j) * D
                            for k in range(KD):
                                accs[k] = accs[k] + \
                                    xbuf[pl.ds(eoff + k * 16, 16)] * wjv
                        offb = s_first * D
                        for k in range(KD):
                            plsc.addupdate(acc.at[pl.ds(offb + k * 16, 16)],
                                           accs[k])

                    @pl.when(s_first != s_last)
                    def _():
                        offv = segv * D
                        for j in range(16):
                            off = offv[j]
                            wj = wv[j]
                            eoff = (e0 + j) * D
                            for k in range(KD):
                                xk = xbuf[pl.ds(eoff + k * 16, 16)]
                                plsc.addupdate(
                                    acc.at[pl.ds(off + k * 16, 16)],
                                    xk * wj)
                    return c2

                lax.fori_loop(0, SUB // 16, blk, 0)

                nci = ci + NBUF

                @pl.when(nci < NSUB)
                def _(b=b, nci=nci):
                    issue(nci, b)

            return c

        lax.fori_loop(0, NOUT, outer, 0)
        pltpu.sync_copy(acc, out_hbm.at[wid])

    return body


def _reduce_partials(p, G, D):
    NW = p.shape[0]

    def body(p_ref, o_ref):
        o_ref[...] = jnp.sum(p_ref[...], axis=0, keepdims=True)

    return pl.pallas_call(
        body,
        out_shape=jax.ShapeDtypeStruct((1, G * D), jnp.float32),
    )(p)


def kernel(edge_feats, segment_ids, num_segments, W, b):
    E, D = edge_feats.shape
    G = 256
    w = _edge_weights(edge_feats, W, b.reshape(1, 1))
    partials = _sc_segsum(E, D, G)(
        edge_feats.reshape(-1), segment_ids, w.reshape(-1)
    )
    h_g_sum = _reduce_partials(partials, G, D).reshape(G, D)
    return (h_g_sum, w)


# speedup vs baseline: 4.1988x; 1.0548x over previous
import functools

import jax
import jax.numpy as jnp
from jax import lax
from jax.experimental import pallas as pl
from jax.experimental.pallas import tpu as pltpu
from jax.experimental.pallas import tpu_sc as plsc


def _edge_weights(x, W, b2):
    E, D = x.shape
    BE = 8000

    def body(x_ref, w_ref, b_ref, o_ref):
        s = jnp.dot(x_ref[...], w_ref[...], preferred_element_type=jnp.float32)
        o_ref[...] = jnp.tanh(s + b_ref[0, 0])

    return pl.pallas_call(
        body,
        grid=(E // BE,),
        in_specs=[
            pl.BlockSpec((BE, D), lambda i: (i, 0)),
            pl.BlockSpec((D, 1), lambda i: (0, 0)),
            pl.BlockSpec((1, 1), lambda i: (0, 0)),
        ],
        out_specs=pl.BlockSpec((BE, 1), lambda i: (i, 0)),
        out_shape=jax.ShapeDtypeStruct((E, 1), jnp.float32),
    )(x, W, b2)


@functools.cache
def _sc_segsum(E, D, G):
    NC, NS = 2, 16
    NW = NC * NS
    EPW = E // NW
    SUB = 80
    NBUF = 5
    NSUB = EPW // SUB
    NOUT = NSUB // NBUF
    GD = G * D
    KD = D // 16

    mesh = plsc.VectorSubcoreMesh(core_axis_name="c", subcore_axis_name="s")

    @functools.partial(
        pl.kernel,
        mesh=mesh,
        out_type=jax.ShapeDtypeStruct((NW, GD), jnp.float32),
        scratch_types=[
            pltpu.VMEM((NBUF * SUB * D,), jnp.float32),
            pltpu.VMEM((NBUF * SUB,), jnp.int32),
            pltpu.VMEM((NBUF * SUB,), jnp.float32),
            pltpu.VMEM((GD,), jnp.float32),
            pltpu.SemaphoreType.DMA,
            pltpu.SemaphoreType.DMA,
            pltpu.SemaphoreType.DMA,
            pltpu.SemaphoreType.DMA,
            pltpu.SemaphoreType.DMA,
        ],
    )
    def body(x_hbm, seg_hbm, w_hbm, out_hbm, xbuf, segb, wbb, acc,
             s0, s1, s2, s3, s4):
        sems = (s0, s1, s2, s3, s4)
        wid = lax.axis_index("c") * NS + lax.axis_index("s")
        base = wid * EPW
        z16 = jnp.zeros((16,), jnp.float32)
        lane = [jnp.full((16,), j, jnp.int32) for j in range(16)]

        def zero_body(i, c):
            acc[pl.ds(i * 16, 16)] = z16
            return c

        lax.fori_loop(0, GD // 16, zero_body, 0)

        def issue(ci, b):
            cb = base + ci * SUB
            pltpu.async_copy(x_hbm.at[pl.ds(cb * D, SUB * D)],
                             xbuf.at[pl.ds(b * SUB * D, SUB * D)], sems[b])
            pltpu.async_copy(seg_hbm.at[pl.ds(cb, SUB)],
                             segb.at[pl.ds(b * SUB, SUB)], sems[b])
            pltpu.async_copy(w_hbm.at[pl.ds(cb, SUB)],
                             wbb.at[pl.ds(b * SUB, SUB)], sems[b])

        def drain(b):
            pltpu.make_async_copy(x_hbm.at[pl.ds(0, SUB * D)],
                                  xbuf.at[pl.ds(b * SUB * D, SUB * D)],
                                  sems[b]).wait()
            pltpu.make_async_copy(seg_hbm.at[pl.ds(0, SUB)],
                                  segb.at[pl.ds(b * SUB, SUB)], sems[b]).wait()
            pltpu.make_async_copy(w_hbm.at[pl.ds(0, SUB)],
                                  wbb.at[pl.ds(b * SUB, SUB)], sems[b]).wait()

        for b in range(NBUF):
            issue(b, b)

        def outer(cg, c):
            for b in range(NBUF):
                ci = cg * NBUF + b
                drain(b)

                def blk(bi, c2, _b=b):
                    e0 = _b * SUB + bi * 16
                    segv = jnp.minimum(segb[pl.ds(e0, 16)], G - 1)
                    wv = wbb[pl.ds(e0, 16)]
                    s_first = segv[0]
                    s_last = segv[15]

                    @pl.when(s_first == s_last)
                    def _():
                        accs = [jnp.zeros((16,), jnp.float32)
                                for _ in range(KD)]
                        for j in range(16):
                            wjv = jnp.take_along_axis(wv, lane[j], axis=0)
                            eoff = (e0 + j) * D
                            for k in range(KD):
                                accs[k] = accs[k] + \
                                    xbuf[pl.ds(eoff + k * 16, 16)] * wjv
                        offb = s_first * D
                        for k in range(KD):
                            plsc.addupdate(acc.at[pl.ds(offb + k * 16, 16)],
                                           accs[k])

                    @pl.when(s_first != s_last)
                    def _():
                        offv = segv * D
                        for j in range(16):
                            off = offv[j]
                            wj = wv[j]
                            eoff = (e0 + j) * D
                            for k in range(KD):
                                xk = xbuf[pl.ds(eoff + k * 16, 16)]
                                plsc.addupdate(
                                    acc.at[pl.ds(off + k * 16, 16)],
                                    xk * wj)
                    return c2

                lax.fori_loop(0, SUB // 16, blk, 0)

                nci = ci + NBUF

                @pl.when(nci < NSUB)
                def _(b=b, nci=nci):
                    issue(nci, b)

            return c

        lax.fori_loop(0, NOUT, outer, 0)
        pltpu.sync_copy(acc, out_hbm.at[wid])

    return body


def _reduce_partials(p, G, D):
    NW = p.shape[0]

    def body(p_ref, o_ref):
        o_ref[...] = jnp.sum(p_ref[...], axis=0, keepdims=True)

    return pl.pallas_call(
        body,
        out_shape=jax.ShapeDtypeStruct((1, G * D), jnp.float32),
    )(p)


def kernel(edge_feats, segment_ids, num_segments, W, b):
    E, D = edge_feats.shape
    G = 256
    w = _edge_weights(edge_feats, W, b.reshape(1, 1))
    partials = _sc_segsum(E, D, G)(
        edge_feats.reshape(-1), segment_ids, w.reshape(-1)
    )
    h_g_sum = _reduce_partials(partials, G, D).reshape(G, D)
    return (h_g_sum, w)
